# baseline (speedup 1.0000x reference)
import functools

import jax
import jax.numpy as jnp
from jax import lax
from jax.experimental import pallas as pl
from jax.experimental.pallas import tpu as pltpu
from jax.experimental.pallas import tpu_sc as plsc

_L = 16
_CH = 4
_NB = 4
_TB = 128


def _shuffle(v, idx):
    dnums = lax.GatherDimensionNumbers(
        offset_dims=(), collapsed_slice_dims=(0,), start_index_map=(0,))
    return lax.gather(v, idx[:, None], dnums, slice_sizes=(1,),
                      mode=lax.GatherScatterMode.PROMISE_IN_BOUNDS)


def _make_score_kernel(B: int, K: int):
    info = plsc.get_sparse_core_info()
    NC, NS = info.num_cores, info.num_subcores
    NW = NC * NS
    assert B % (NW * _NB * _CH) == 0 and K == _L
    bpw = B // NW
    nchunk = bpw // _CH
    nsuper = nchunk // _NB

    mesh = plsc.VectorSubcoreMesh(core_axis_name="c", subcore_axis_name="s")

    @functools.partial(
        pl.kernel,
        mesh=mesh,
        out_type=jax.ShapeDtypeStruct((B,), jnp.float32),
        scratch_types=[
            pltpu.VMEM((bpw + 4 * _L,), jnp.int32),
            pltpu.VMEM((bpw + 4 * _L,), jnp.int32),
            pltpu.VMEM((_NB, _CH, _L, _TB), jnp.float32),
            pltpu.VMEM((_NB, _CH, _L, _TB), jnp.float32),
            pltpu.VMEM((bpw,), jnp.float32),
            pltpu.SemaphoreType.DMA,
            pltpu.SemaphoreType.DMA,
            pltpu.SemaphoreType.DMA,
            pltpu.SemaphoreType.DMA,
        ],
    )
    def score(uidx_hbm, iidx_hbm, wt_hbm, ht_hbm, out_hbm,
              u_sm, i_sm, ublk_v, vblk_v, out_v, *sems):
        wid = lax.axis_index("s") * NC + lax.axis_index("c")
        base = wid * bpw
        pltpu.sync_copy(uidx_hbm.at[pl.ds(base, bpw)], u_sm.at[pl.ds(0, bpw)])
        pltpu.sync_copy(iidx_hbm.at[pl.ds(base, bpw)], i_sm.at[pl.ds(0, bpw)])

        lanes = lax.iota(jnp.int32, _L)

        def fire(c, buf, sem):
            uvec = u_sm[pl.ds(c * _CH, _L)]
            ivec = i_sm[pl.ds(c * _CH, _L)]
            for j in range(_CH):
                bu = pl.multiple_of((uvec[j] >> 7) * _TB, _TB)
                bi = pl.multiple_of((ivec[j] >> 7) * _TB, _TB)
                pltpu.async_copy(
                    wt_hbm.at[:, pl.ds(bu, _TB)], ublk_v.at[buf, j], sem)
                pltpu.async_copy(
                    ht_hbm.at[:, pl.ds(bi, _TB)], vblk_v.at[buf, j], sem)

        def drain(buf, sem):
            dummy = wt_hbm.at[:, pl.ds(0, _TB)]
            for j in range(_CH):
                pltpu.make_async_copy(dummy, ublk_v.at[buf, j], sem).wait()
                pltpu.make_async_copy(dummy, vblk_v.at[buf, j], sem).wait()

        def score_chunk(c, buf, lane0, acc):
            uvec = u_sm[pl.ds(c * _CH, _L)]
            ivec = i_sm[pl.ds(c * _CH, _L)]
            for j in range(_CH):
                mu = uvec[j] & 127
                mi = ivec[j] & 127
                mua = (mu >> 4) << 4
                mia = (mi >> 4) << 4
                mu15 = mu & 15
                mi15 = mi & 15
                rot = (lanes + (mi15 - mu15)) & 15
                ps = jnp.zeros((_L,), jnp.float32)
                for k in range(_L):
                    ua = ublk_v[buf, j, k, pl.ds(mua, _L)]
                    va = vblk_v[buf, j, k, pl.ds(mia, _L)]
                    ps = ps + ua * _shuffle(va, rot)
                dot = _shuffle(ps, jnp.broadcast_to(mu15, (_L,)))
                acc = jnp.where(lanes == lane0 + j, dot, acc)
            return acc

        for b in range(_NB):
            fire(b, b, sems[b])

        def super_body(g, carry):
            c0 = _NB * g
            acc = jnp.zeros((_L,), jnp.float32)
            for b in range(_NB):
                drain(b, sems[b])
                acc = score_chunk(c0 + b, b, b * _CH, acc)

                @pl.when(c0 + _NB + b < nchunk)
                def _():
                    fire(c0 + _NB + b, b, sems[b])

            sig = 1.0 / (1.0 + jnp.exp(-acc))
            out_v[pl.ds(c0 * _CH, _L)] = sig
            return carry

        lax.fori_loop(0, nsuper, super_body, 0)
        pltpu.sync_copy(out_v, out_hbm.at[pl.ds(base, bpw)])

    return score


def kernel(x, W, H):
    x = x.astype(jnp.int32)
    B = x.shape[0]
    score = _make_score_kernel(B, W.shape[1])
    return score(x[:, 0], x[:, 1], W.T, H.T)


# speedup vs baseline: 1.0897x; 1.0897x over previous
import functools

import jax
import jax.numpy as jnp
from jax import lax
from jax.experimental import pallas as pl
from jax.experimental.pallas import tpu as pltpu
from jax.experimental.pallas import tpu_sc as plsc

_L = 16
_CH = 2
_NB = 8
_TB = 128


def _shuffle(v, idx):
    dnums = lax.GatherDimensionNumbers(
        offset_dims=(), collapsed_slice_dims=(0,), start_index_map=(0,))
    return lax.gather(v, idx[:, None], dnums, slice_sizes=(1,),
                      mode=lax.GatherScatterMode.PROMISE_IN_BOUNDS)


def _make_score_kernel(B: int, K: int):
    info = plsc.get_sparse_core_info()
    NC, NS = info.num_cores, info.num_subcores
    NW = NC * NS
    assert B % (NW * _NB * _CH) == 0 and K == _L
    bpw = B // NW
    nchunk = bpw // _CH
    nsuper = nchunk // _NB

    mesh = plsc.VectorSubcoreMesh(core_axis_name="c", subcore_axis_name="s")

    @functools.partial(
        pl.kernel,
        mesh=mesh,
        out_type=jax.ShapeDtypeStruct((B,), jnp.float32),
        scratch_types=[
            pltpu.VMEM((bpw + 4 * _L,), jnp.int32),
            pltpu.VMEM((bpw + 4 * _L,), jnp.int32),
            pltpu.VMEM((_NB, _CH, _L, _TB), jnp.float32),
            pltpu.VMEM((_NB, _CH, _L, _TB), jnp.float32),
            pltpu.VMEM((bpw,), jnp.float32),
        ] + [pltpu.SemaphoreType.DMA] * _NB,
    )
    def score(uidx_hbm, iidx_hbm, wt_hbm, ht_hbm, out_hbm,
              u_sm, i_sm, ublk_v, vblk_v, out_v, *sems):
        wid = lax.axis_index("s") * NC + lax.axis_index("c")
        base = wid * bpw
        pltpu.sync_copy(uidx_hbm.at[pl.ds(base, bpw)], u_sm.at[pl.ds(0, bpw)])
        pltpu.sync_copy(iidx_hbm.at[pl.ds(base, bpw)], i_sm.at[pl.ds(0, bpw)])

        lanes = lax.iota(jnp.int32, _L)

        def fire(c, buf, sem):
            uvec = u_sm[pl.ds(c * _CH, _L)]
            ivec = i_sm[pl.ds(c * _CH, _L)]
            for j in range(_CH):
                bu = pl.multiple_of((uvec[j] >> 7) * _TB, _TB)
                bi = pl.multiple_of((ivec[j] >> 7) * _TB, _TB)
                pltpu.async_copy(
                    wt_hbm.at[:, pl.ds(bu, _TB)], ublk_v.at[buf, j], sem)
                pltpu.async_copy(
                    ht_hbm.at[:, pl.ds(bi, _TB)], vblk_v.at[buf, j], sem)

        def drain(buf, sem):
            dummy = wt_hbm.at[:, pl.ds(0, _TB)]
            for j in range(_CH):
                pltpu.make_async_copy(dummy, ublk_v.at[buf, j], sem).wait()
                pltpu.make_async_copy(dummy, vblk_v.at[buf, j], sem).wait()

        def score_chunk(c, buf, lane0, acc):
            uvec = u_sm[pl.ds(c * _CH, _L)]
            ivec = i_sm[pl.ds(c * _CH, _L)]
            for j in range(_CH):
                mu = uvec[j] & 127
                mi = ivec[j] & 127
                mua = (mu >> 4) << 4
                mia = (mi >> 4) << 4
                mu15 = mu & 15
                mi15 = mi & 15
                rot = (lanes + (mi15 - mu15)) & 15
                ps = jnp.zeros((_L,), jnp.float32)
                for k in range(_L):
                    ua = ublk_v[buf, j, k, pl.ds(mua, _L)]
                    va = vblk_v[buf, j, k, pl.ds(mia, _L)]
                    ps = ps + ua * _shuffle(va, rot)
                dot = _shuffle(ps, jnp.broadcast_to(mu15, (_L,)))
                acc = jnp.where(lanes == lane0 + j, dot, acc)
            return acc

        for b in range(_NB):
            fire(b, b, sems[b])

        def super_body(g, carry):
            c0 = _NB * g
            acc = jnp.zeros((_L,), jnp.float32)
            for b in range(_NB):
                drain(b, sems[b])
                acc = score_chunk(c0 + b, b, b * _CH, acc)

                @pl.when(c0 + _NB + b < nchunk)
                def _():
                    fire(c0 + _NB + b, b, sems[b])

            sig = 1.0 / (1.0 + jnp.exp(-acc))
            out_v[pl.ds(c0 * _CH, _L)] = sig
            return carry

        lax.fori_loop(0, nsuper, super_body, 0)
        pltpu.sync_copy(out_v, out_hbm.at[pl.ds(base, bpw)])

    return score


def kernel(x, W, H):
    x = x.astype(jnp.int32)
    B = x.shape[0]
    score = _make_score_kernel(B, W.shape[1])
    return score(x[:, 0], x[:, 1], W.T, H.T)
